# baseline (speedup 1.0000x reference)
import functools

import jax
import jax.numpy as jnp
from jax import lax
from jax.experimental import pallas as pl
from jax.experimental.pallas import tpu as pltpu
from jax.experimental.pallas import tpu_sc as plsc

N = 10000
E = 160000
D = 128
NC = 2
NS = 16
NW = NC * NS
EPW = E // NW
CH = 128
NCH = 40
PADE = NCH * CH
ND = N + 8
RPT = N // NS

@functools.lru_cache(maxsize=None)
def _sc_kernels():
    mesh = plsc.VectorSubcoreMesh(core_axis_name="c", subcore_axis_name="s")

    @functools.partial(
        pl.kernel,
        mesh=mesh,
        out_type=jax.ShapeDtypeStruct((NC * 4 * N,), jnp.float32),
        scratch_types=[
            pltpu.VMEM((NCH, CH), jnp.int32),
            pltpu.VMEM((NCH, CH), jnp.int32),
            pltpu.VMEM((NCH, CH), jnp.int32),
            pltpu.VMEM((NCH, CH), jnp.int32),
            pltpu.VMEM((CH,), jnp.float32),
            pltpu.VMEM((1000,), jnp.float32),
            pltpu.VMEM_SHARED((ND,), jnp.float32),
            pltpu.VMEM_SHARED((ND,), jnp.float32),
            pltpu.VMEM_SHARED((ND,), jnp.float32),
            pltpu.VMEM_SHARED((ND,), jnp.float32),
            pltpu.SemaphoreType.DMA,
        ],
    )
    def deg_kernel(src_ln, dst_ln, src_nl, dst_nl, ones_hbm, zeros_hbm, out,
                   i0, i1, i2, i3, ones_v, stage, d0, d1, d2, d3, sem):
        c = lax.axis_index("c")
        s = lax.axis_index("s")
        wid = c * NS + s
        degs = (d0, d1, d2, d3)
        idxs = (i0, i1, i2, i3)
        pltpu.sync_copy(ones_hbm, ones_v)
        for k, idx_hbm in enumerate((src_ln, dst_ln, src_nl, dst_nl)):
            pltpu.sync_copy(idx_hbm.at[wid], idxs[k])

        @pl.when(s < 10)
        def _():
            pltpu.sync_copy(zeros_hbm.at[pl.ds(s * 1000, 1000)], stage)
            for k in range(4):
                pltpu.sync_copy(stage, degs[k].at[pl.ds(s * 1000, 1000)])

        plsc.subcore_barrier()

        for k in range(4):
            def fire(j, _, k=k):
                pltpu.async_copy(ones_v, degs[k].at[idxs[k].at[j]], sem,
                                 add=True)
                return _
            lax.fori_loop(0, NCH, fire, 0)
            def drain(j, _, k=k):
                pltpu.make_async_copy(ones_v, degs[k].at[idxs[k].at[0]],
                                      sem).wait()
                return _
            lax.fori_loop(0, NCH, drain, 0)

        plsc.subcore_barrier()

        @pl.when(s < 10)
        def _():
            for k in range(4):
                pltpu.sync_copy(degs[k].at[pl.ds(s * 1000, 1000)], stage)
                pltpu.sync_copy(
                    stage,
                    out.at[pl.ds(c * 4 * N + k * N + s * 1000, 1000)])

    @functools.partial(
        pl.kernel,
        mesh=mesh,
        out_type=jax.ShapeDtypeStruct((NC * N, D), jnp.float32),
        scratch_types=[
            pltpu.VMEM((CH,), jnp.int32),
            pltpu.VMEM((CH,), jnp.int32),
            pltpu.VMEM((CH,), jnp.int32),
            pltpu.VMEM((CH,), jnp.int32),
            pltpu.VMEM((CH, D), jnp.float32),
            pltpu.VMEM((CH, D), jnp.float32),
            pltpu.VMEM_SHARED((ND, D), jnp.float32),
            pltpu.SemaphoreType.DMA,
            pltpu.SemaphoreType.DMA,
        ],
    )
    def edge_kernel(h_hbm, srcp_hbm, dstp_hbm, zeros_hbm, out,
                    idx_s0, idx_d0, idx_s1, idx_d1, rows0, rows1,
                    agg, sg0, sg1):
        c = lax.axis_index("c")
        s = lax.axis_index("s")
        wid = c * NS + s
        bufs = (rows0, rows1)
        gsems = (sg0, sg1)
        sidx = (idx_s0, idx_s1)
        didx = (idx_d0, idx_d1)

        pltpu.sync_copy(zeros_hbm.at[pl.ds(0, CH)], rows0)
        for t in range(5):
            pltpu.async_copy(rows0.at[pl.ds(0, 125)],
                             agg.at[pl.ds(s * RPT + t * 125, 125)], sg0)
        for t in range(5):
            pltpu.make_async_copy(rows0.at[pl.ds(0, 125)],
                                  agg.at[pl.ds(s * RPT + t * 125, 125)],
                                  sg0).wait()
        plsc.subcore_barrier()

        pltpu.sync_copy(srcp_hbm.at[wid, 0], idx_s0)
        pltpu.sync_copy(dstp_hbm.at[wid, 0], idx_d0)
        pltpu.async_copy(h_hbm.at[idx_s0], rows0, sg0)

        def pair(g, carry):
            for b in range(2):
                j = 2 * g + b

                @pl.when(j + 1 < NCH)
                def _(b=b, j=j):
                    pltpu.sync_copy(srcp_hbm.at[wid, j + 1], sidx[1 - b])
                    pltpu.sync_copy(dstp_hbm.at[wid, j + 1], didx[1 - b])
                    pltpu.async_copy(h_hbm.at[sidx[1 - b]], bufs[1 - b],
                                     gsems[1 - b])

                pltpu.make_async_copy(h_hbm.at[sidx[b]], bufs[b],
                                      gsems[b]).wait()
                pltpu.sync_copy(bufs[b], agg.at[didx[b]], add=True)
            return carry
        lax.fori_loop(0, NCH // 2, pair, 0)

        plsc.subcore_barrier()

        sizes = [CH] * 7 + [1000 - 7 * CH]

        @pl.when(s < 10)
        def _():
            pltpu.async_copy(agg.at[pl.ds(s * 1000, sizes[0])],
                             rows0.at[pl.ds(0, sizes[0])], sg0)
            for t in range(8):
                buf, sem = bufs[t % 2], gsems[t % 2]
                r0 = s * 1000 + t * CH
                pltpu.make_async_copy(agg.at[pl.ds(r0, sizes[t])],
                                      buf.at[pl.ds(0, sizes[t])], sem).wait()
                if t + 1 < 8:
                    pltpu.async_copy(
                        agg.at[pl.ds(r0 + CH, sizes[t + 1])],
                        bufs[(t + 1) % 2].at[pl.ds(0, sizes[t + 1])],
                        gsems[(t + 1) % 2])
                pltpu.sync_copy(buf.at[pl.ds(0, sizes[t])],
                                out.at[pl.ds(c * N + r0, sizes[t])])

    return deg_kernel, edge_kernel


_BM = 1000


def _mm_body(x_ref, w_ref, s_ref, o_ref):
    o_ref[...] = (
        jnp.dot(x_ref[...], w_ref[...], preferred_element_type=jnp.float32)
        * s_ref[...]
    )


def _matmul_scaled(x, w, scale):
    k = x.shape[1]
    return pl.pallas_call(
        _mm_body,
        grid=(N // _BM,),
        in_specs=[
            pl.BlockSpec((_BM, k), lambda i: (i, 0)),
            pl.BlockSpec((k, D), lambda i: (0, 0)),
            pl.BlockSpec((_BM, 1), lambda i: (i, 0)),
        ],
        out_specs=pl.BlockSpec((_BM, D), lambda i: (i, 0)),
        out_shape=jax.ShapeDtypeStruct((N, D), jnp.float32),
    )(x, w, scale)


def _scale_body(d_ref, o_ref):
    t = d_ref[0] + d_ref[1]
    o_ref[...] = lax.rsqrt(jnp.maximum(t, 1.0))


def _deg_scales(deg_partials):
    return pl.pallas_call(
        _scale_body,
        out_shape=jax.ShapeDtypeStruct((4, N), jnp.float32),
    )(deg_partials)


def _post_body(p_ref, s_ref, b_ref, g_ref, bb_ref, o_ref):
    a = p_ref[0] + p_ref[1]
    a = a * s_ref[...] + b_ref[...]
    mu = jnp.mean(a, axis=1, keepdims=True)
    d = a - mu
    var = jnp.mean(d * d, axis=1, keepdims=True)
    y = d * lax.rsqrt(var + 1e-5) * g_ref[...] + bb_ref[...]
    o_ref[...] = jnp.where(y > 0, y, jnp.exp(jnp.minimum(y, 0.0)) - 1.0)


def _post(partials, in_scale, bias, gamma, beta):
    return pl.pallas_call(
        _post_body,
        grid=(N // _BM,),
        in_specs=[
            pl.BlockSpec((2, _BM, D), lambda i: (0, i, 0)),
            pl.BlockSpec((_BM, 1), lambda i: (i, 0)),
            pl.BlockSpec((1, D), lambda i: (0, 0)),
            pl.BlockSpec((1, D), lambda i: (0, 0)),
            pl.BlockSpec((1, D), lambda i: (0, 0)),
        ],
        out_specs=pl.BlockSpec((_BM, D), lambda i: (i, 0)),
        out_shape=jax.ShapeDtypeStruct((N, D), jnp.float32),
    )(partials, in_scale, bias, gamma, beta)


def _readout_body(hn_ref, hl_ref, wfc_ref, bfc_ref, wout_ref, bout_ref,
                  o_ref, acc_ref):
    i = pl.program_id(0)

    @pl.when(i == 0)
    def _():
        acc_ref[...] = jnp.zeros_like(acc_ref)

    acc_ref[0:1] += jnp.sum(hn_ref[...] + hl_ref[...], axis=0, keepdims=True)

    @pl.when(i == pl.num_programs(0) - 1)
    def _():
        hg = acc_ref[0:1] * (1.0 / N)
        z = jnp.maximum(
            jnp.dot(hg, wfc_ref[...], preferred_element_type=jnp.float32)
            + bfc_ref[...], 0.0)
        o_ref[...] = (
            jnp.dot(z, wout_ref[...], preferred_element_type=jnp.float32)
            + bout_ref[...])


def _readout(h_n, h_l, w_fc, b_fc, w_out, b_out):
    return pl.pallas_call(
        _readout_body,
        grid=(N // _BM,),
        in_specs=[
            pl.BlockSpec((_BM, D), lambda i: (i, 0)),
            pl.BlockSpec((_BM, D), lambda i: (i, 0)),
            pl.BlockSpec((D, D), lambda i: (0, 0)),
            pl.BlockSpec((1, D), lambda i: (0, 0)),
            pl.BlockSpec((D, 1), lambda i: (0, 0)),
            pl.BlockSpec((1, 1), lambda i: (0, 0)),
        ],
        out_specs=pl.BlockSpec((1, 1), lambda i: (0, 0)),
        out_shape=jax.ShapeDtypeStruct((1, 1), jnp.float32),
        scratch_shapes=[pltpu.VMEM((8, D), jnp.float32)],
    )(h_n, h_l, w_fc, b_fc, w_out, b_out)


def _pad_k(x, mult=128):
    k = x.shape[1]
    kp = ((k + mult - 1) // mult) * mult
    if kp == k:
        return x
    return jnp.pad(x, ((0, 0), (0, kp - k)))


def _pad_idx(a, fill):
    per = a.reshape(NW, EPW)
    pad = jnp.full((NW, PADE - EPW), fill, jnp.int32)
    return jnp.concatenate([per, pad], axis=1).reshape(NW, NCH, CH)


def kernel(feat_n, feat_l, edge_index_l2n, edge_index_n2l, params):
    src_ln = edge_index_l2n[0].astype(jnp.int32)
    dst_ln = edge_index_l2n[1].astype(jnp.int32)
    src_nl = edge_index_n2l[0].astype(jnp.int32)
    dst_nl = edge_index_n2l[1].astype(jnp.int32)

    srcp_ln = _pad_idx(src_ln, 0)
    dstp_ln = _pad_idx(dst_ln, N)
    srcp_nl = _pad_idx(src_nl, 0)
    dstp_nl = _pad_idx(dst_nl, N)
    dpad = [_pad_idx(a, N) for a in (src_ln, dst_ln, src_nl, dst_nl)]

    ones_v = jnp.ones((CH,), jnp.float32)
    zeros_n = jnp.zeros((N,), jnp.float32)
    zeros_nd = jnp.zeros((N, D), jnp.float32)

    _deg_kernel, _edge_kernel = _sc_kernels()
    deg_flat = _deg_kernel(dpad[0], dpad[1], dpad[2], dpad[3], ones_v, zeros_n)
    scales = _deg_scales(deg_flat.reshape(NC, 4, N))
    out_l = scales[0][:, None]
    in_n = scales[1][:, None]
    out_n = scales[2][:, None]
    in_l = scales[3][:, None]

    h_n, h_l = feat_n, feat_l
    for i in range(3):
        w_ln = params['W_l2n'][i]
        w_nl = params['W_n2l'][i]
        x_l = _pad_k(h_l)
        x_n = _pad_k(h_n)
        w_ln = jnp.pad(w_ln, ((0, x_l.shape[1] - w_ln.shape[0]), (0, 0)))
        w_nl = jnp.pad(w_nl, ((0, x_n.shape[1] - w_nl.shape[0]), (0, 0)))

        m_ln = _matmul_scaled(x_l, w_ln, out_l)
        m_nl = _matmul_scaled(x_n, w_nl, out_n)

        p_ln = _edge_kernel(m_ln, srcp_ln, dstp_ln, zeros_nd).reshape(NC, N, D)
        p_nl = _edge_kernel(m_nl, srcp_nl, dstp_nl, zeros_nd).reshape(NC, N, D)

        h_n = _post(p_ln, in_n, params['b_l2n'][i][None, :],
                    params['ln_n_g'][i][None, :], params['ln_n_b'][i][None, :])
        h_l = _post(p_nl, in_l, params['b_n2l'][i][None, :],
                    params['ln_l_g'][i][None, :], params['ln_l_b'][i][None, :])

    return _readout(h_n, h_l, params['W_fc'], params['b_fc'][None, :],
                    params['W_out'], params['b_out'][None, :])


# speedup vs baseline: 2.0500x; 2.0500x over previous
import functools

import jax
import jax.numpy as jnp
from jax import lax
from jax.experimental import pallas as pl
from jax.experimental.pallas import tpu as pltpu
from jax.experimental.pallas import tpu_sc as plsc

N = 10000
E = 160000
D = 128
NC = 2
NS = 16
NW = NC * NS
EPW = E // NW
CH = 128
NCH = 40
PADE = NCH * CH
ND = N + NW
RPT = N // NS

@functools.lru_cache(maxsize=None)
def _sc_kernels():
    mesh = plsc.VectorSubcoreMesh(core_axis_name="c", subcore_axis_name="s")

    @functools.partial(
        pl.kernel,
        mesh=mesh,
        out_type=jax.ShapeDtypeStruct((NC * 4 * N,), jnp.float32),
        scratch_types=[
            pltpu.VMEM((NCH, CH), jnp.int32),
            pltpu.VMEM((NCH, CH), jnp.int32),
            pltpu.VMEM((NCH, CH), jnp.int32),
            pltpu.VMEM((NCH, CH), jnp.int32),
            pltpu.VMEM((CH,), jnp.float32),
            pltpu.VMEM((1000,), jnp.float32),
            pltpu.VMEM_SHARED((ND,), jnp.float32),
            pltpu.VMEM_SHARED((ND,), jnp.float32),
            pltpu.VMEM_SHARED((ND,), jnp.float32),
            pltpu.VMEM_SHARED((ND,), jnp.float32),
            pltpu.SemaphoreType.DMA,
        ],
    )
    def deg_kernel(src_ln, dst_ln, src_nl, dst_nl, ones_hbm, zeros_hbm, out,
                   i0, i1, i2, i3, ones_v, stage, d0, d1, d2, d3, sem):
        c = lax.axis_index("c")
        s = lax.axis_index("s")
        wid = c * NS + s
        degs = (d0, d1, d2, d3)
        idxs = (i0, i1, i2, i3)
        pltpu.sync_copy(ones_hbm, ones_v)
        for k, idx_hbm in enumerate((src_ln, dst_ln, src_nl, dst_nl)):
            pltpu.sync_copy(idx_hbm.at[wid], idxs[k])

        @pl.when(s < 10)
        def _():
            pltpu.sync_copy(zeros_hbm.at[pl.ds(s * 1000, 1000)], stage)
            for k in range(4):
                pltpu.sync_copy(stage, degs[k].at[pl.ds(s * 1000, 1000)])

        plsc.subcore_barrier()

        for k in range(4):
            def fire(j, _, k=k):
                pltpu.async_copy(ones_v, degs[k].at[idxs[k].at[j]], sem,
                                 add=True)
                return _
            lax.fori_loop(0, NCH, fire, 0)
            def drain(j, _, k=k):
                pltpu.make_async_copy(ones_v, degs[k].at[idxs[k].at[0]],
                                      sem).wait()
                return _
            lax.fori_loop(0, NCH, drain, 0)

        plsc.subcore_barrier()

        @pl.when(s < 10)
        def _():
            for k in range(4):
                pltpu.sync_copy(degs[k].at[pl.ds(s * 1000, 1000)], stage)
                pltpu.sync_copy(
                    stage,
                    out.at[pl.ds(c * 4 * N + k * N + s * 1000, 1000)])

    @functools.partial(
        pl.kernel,
        mesh=mesh,
        out_type=jax.ShapeDtypeStruct((NC * N, D), jnp.float32),
        scratch_types=[
            pltpu.VMEM((CH,), jnp.int32),
            pltpu.VMEM((CH,), jnp.int32),
            pltpu.VMEM((CH,), jnp.int32),
            pltpu.VMEM((CH,), jnp.int32),
            pltpu.VMEM((CH, D), jnp.float32),
            pltpu.VMEM((CH, D), jnp.float32),
            pltpu.VMEM_SHARED((ND, D), jnp.float32),
            pltpu.SemaphoreType.DMA,
            pltpu.SemaphoreType.DMA,
        ],
    )
    def edge_kernel(h_hbm, srcp_hbm, dstp_hbm, zeros_hbm, out,
                    idx_s0, idx_d0, idx_s1, idx_d1, rows0, rows1,
                    agg, sg0, sg1):
        c = lax.axis_index("c")
        s = lax.axis_index("s")
        wid = c * NS + s
        bufs = (rows0, rows1)
        gsems = (sg0, sg1)
        sidx = (idx_s0, idx_s1)
        didx = (idx_d0, idx_d1)

        pltpu.sync_copy(zeros_hbm.at[pl.ds(0, CH)], rows0)
        for t in range(5):
            pltpu.async_copy(rows0.at[pl.ds(0, 125)],
                             agg.at[pl.ds(s * RPT + t * 125, 125)], sg0)
        for t in range(5):
            pltpu.make_async_copy(rows0.at[pl.ds(0, 125)],
                                  agg.at[pl.ds(s * RPT + t * 125, 125)],
                                  sg0).wait()
        plsc.subcore_barrier()

        pltpu.sync_copy(srcp_hbm.at[wid, 0], idx_s0)
        pltpu.sync_copy(dstp_hbm.at[wid, 0], idx_d0)
        pltpu.async_copy(h_hbm.at[idx_s0], rows0, sg0)

        def pair(g, carry):
            for b in range(2):
                j = 2 * g + b

                @pl.when(j + 1 < NCH)
                def _(b=b, j=j):
                    pltpu.sync_copy(srcp_hbm.at[wid, j + 1], sidx[1 - b])
                    pltpu.sync_copy(dstp_hbm.at[wid, j + 1], didx[1 - b])
                    pltpu.async_copy(h_hbm.at[sidx[1 - b]], bufs[1 - b],
                                     gsems[1 - b])

                pltpu.make_async_copy(h_hbm.at[sidx[b]], bufs[b],
                                      gsems[b]).wait()
                pltpu.sync_copy(bufs[b], agg.at[didx[b]], add=True)
            return carry
        lax.fori_loop(0, NCH // 2, pair, 0)

        plsc.subcore_barrier()

        sizes = [CH] * 7 + [1000 - 7 * CH]

        @pl.when(s < 10)
        def _():
            pltpu.async_copy(agg.at[pl.ds(s * 1000, sizes[0])],
                             rows0.at[pl.ds(0, sizes[0])], sg0)
            for t in range(8):
                buf, sem = bufs[t % 2], gsems[t % 2]
                r0 = s * 1000 + t * CH
                pltpu.make_async_copy(agg.at[pl.ds(r0, sizes[t])],
                                      buf.at[pl.ds(0, sizes[t])], sem).wait()
                if t + 1 < 8:
                    pltpu.async_copy(
                        agg.at[pl.ds(r0 + CH, sizes[t + 1])],
                        bufs[(t + 1) % 2].at[pl.ds(0, sizes[t + 1])],
                        gsems[(t + 1) % 2])
                pltpu.sync_copy(buf.at[pl.ds(0, sizes[t])],
                                out.at[pl.ds(c * N + r0, sizes[t])])

    return deg_kernel, edge_kernel


_BM = 1000


def _mm_body(x_ref, w_ref, s_ref, o_ref):
    o_ref[...] = (
        jnp.dot(x_ref[...], w_ref[...], preferred_element_type=jnp.float32)
        * s_ref[...]
    )


def _matmul_scaled(x, w, scale):
    k = x.shape[1]
    return pl.pallas_call(
        _mm_body,
        grid=(N // _BM,),
        in_specs=[
            pl.BlockSpec((_BM, k), lambda i: (i, 0)),
            pl.BlockSpec((k, D), lambda i: (0, 0)),
            pl.BlockSpec((_BM, 1), lambda i: (i, 0)),
        ],
        out_specs=pl.BlockSpec((_BM, D), lambda i: (i, 0)),
        out_shape=jax.ShapeDtypeStruct((N, D), jnp.float32),
    )(x, w, scale)


def _scale_body(d_ref, o_ref):
    t = d_ref[0] + d_ref[1]
    o_ref[...] = lax.rsqrt(jnp.maximum(t, 1.0))


def _deg_scales(deg_partials):
    return pl.pallas_call(
        _scale_body,
        out_shape=jax.ShapeDtypeStruct((4, N), jnp.float32),
    )(deg_partials)


def _post_body(p_ref, s_ref, b_ref, g_ref, bb_ref, o_ref):
    a = p_ref[0] + p_ref[1]
    a = a * s_ref[...] + b_ref[...]
    mu = jnp.mean(a, axis=1, keepdims=True)
    d = a - mu
    var = jnp.mean(d * d, axis=1, keepdims=True)
    y = d * lax.rsqrt(var + 1e-5) * g_ref[...] + bb_ref[...]
    o_ref[...] = jnp.where(y > 0, y, jnp.exp(jnp.minimum(y, 0.0)) - 1.0)


def _post(partials, in_scale, bias, gamma, beta):
    return pl.pallas_call(
        _post_body,
        grid=(N // _BM,),
        in_specs=[
            pl.BlockSpec((2, _BM, D), lambda i: (0, i, 0)),
            pl.BlockSpec((_BM, 1), lambda i: (i, 0)),
            pl.BlockSpec((1, D), lambda i: (0, 0)),
            pl.BlockSpec((1, D), lambda i: (0, 0)),
            pl.BlockSpec((1, D), lambda i: (0, 0)),
        ],
        out_specs=pl.BlockSpec((_BM, D), lambda i: (i, 0)),
        out_shape=jax.ShapeDtypeStruct((N, D), jnp.float32),
    )(partials, in_scale, bias, gamma, beta)


def _readout_body(hn_ref, hl_ref, wfc_ref, bfc_ref, wout_ref, bout_ref,
                  o_ref, acc_ref):
    i = pl.program_id(0)

    @pl.when(i == 0)
    def _():
        acc_ref[...] = jnp.zeros_like(acc_ref)

    acc_ref[0:1] += jnp.sum(hn_ref[...] + hl_ref[...], axis=0, keepdims=True)

    @pl.when(i == pl.num_programs(0) - 1)
    def _():
        hg = acc_ref[0:1] * (1.0 / N)
        z = jnp.maximum(
            jnp.dot(hg, wfc_ref[...], preferred_element_type=jnp.float32)
            + bfc_ref[...], 0.0)
        o_ref[...] = (
            jnp.dot(z, wout_ref[...], preferred_element_type=jnp.float32)
            + bout_ref[...])


def _readout(h_n, h_l, w_fc, b_fc, w_out, b_out):
    return pl.pallas_call(
        _readout_body,
        grid=(N // _BM,),
        in_specs=[
            pl.BlockSpec((_BM, D), lambda i: (i, 0)),
            pl.BlockSpec((_BM, D), lambda i: (i, 0)),
            pl.BlockSpec((D, D), lambda i: (0, 0)),
            pl.BlockSpec((1, D), lambda i: (0, 0)),
            pl.BlockSpec((D, 1), lambda i: (0, 0)),
            pl.BlockSpec((1, 1), lambda i: (0, 0)),
        ],
        out_specs=pl.BlockSpec((1, 1), lambda i: (0, 0)),
        out_shape=jax.ShapeDtypeStruct((1, 1), jnp.float32),
        scratch_shapes=[pltpu.VMEM((8, D), jnp.float32)],
    )(h_n, h_l, w_fc, b_fc, w_out, b_out)


def _pad_k(x, mult=128):
    k = x.shape[1]
    kp = ((k + mult - 1) // mult) * mult
    if kp == k:
        return x
    return jnp.pad(x, ((0, 0), (0, kp - k)))


def _pad_idx(a, dump):
    per = a.reshape(NW, EPW)
    fill = (dump + jnp.arange(NW, dtype=jnp.int32))[:, None]
    pad = jnp.broadcast_to(fill, (NW, PADE - EPW))
    return jnp.concatenate([per, pad], axis=1).reshape(NW, NCH, CH)


def kernel(feat_n, feat_l, edge_index_l2n, edge_index_n2l, params):
    src_ln = edge_index_l2n[0].astype(jnp.int32)
    dst_ln = edge_index_l2n[1].astype(jnp.int32)
    src_nl = edge_index_n2l[0].astype(jnp.int32)
    dst_nl = edge_index_n2l[1].astype(jnp.int32)

    srcp_ln = _pad_idx(src_ln, 0)
    dstp_ln = _pad_idx(dst_ln, N)
    srcp_nl = _pad_idx(src_nl, 0)
    dstp_nl = _pad_idx(dst_nl, N)
    dpad = [_pad_idx(a, N) for a in (src_ln, dst_ln, src_nl, dst_nl)]

    ones_v = jnp.ones((CH,), jnp.float32)
    zeros_n = jnp.zeros((N,), jnp.float32)
    zeros_nd = jnp.zeros((N, D), jnp.float32)

    _deg_kernel, _edge_kernel = _sc_kernels()
    deg_flat = _deg_kernel(dpad[0], dpad[1], dpad[2], dpad[3], ones_v, zeros_n)
    scales = _deg_scales(deg_flat.reshape(NC, 4, N))
    out_l = scales[0][:, None]
    in_n = scales[1][:, None]
    out_n = scales[2][:, None]
    in_l = scales[3][:, None]

    h_n, h_l = feat_n, feat_l
    for i in range(3):
        w_ln = params['W_l2n'][i]
        w_nl = params['W_n2l'][i]
        x_l = _pad_k(h_l)
        x_n = _pad_k(h_n)
        w_ln = jnp.pad(w_ln, ((0, x_l.shape[1] - w_ln.shape[0]), (0, 0)))
        w_nl = jnp.pad(w_nl, ((0, x_n.shape[1] - w_nl.shape[0]), (0, 0)))

        m_ln = _matmul_scaled(x_l, w_ln, out_l)
        m_nl = _matmul_scaled(x_n, w_nl, out_n)

        p_ln = _edge_kernel(m_ln, srcp_ln, dstp_ln, zeros_nd).reshape(NC, N, D)
        p_nl = _edge_kernel(m_nl, srcp_nl, dstp_nl, zeros_nd).reshape(NC, N, D)

        h_n = _post(p_ln, in_n, params['b_l2n'][i][None, :],
                    params['ln_n_g'][i][None, :], params['ln_n_b'][i][None, :])
        h_l = _post(p_nl, in_l, params['b_n2l'][i][None, :],
                    params['ln_l_g'][i][None, :], params['ln_l_b'][i][None, :])

    return _readout(h_n, h_l, params['W_fc'], params['b_fc'][None, :],
                    params['W_out'], params['b_out'][None, :])
